# baseline (speedup 1.0000x reference)
import jax
import jax.numpy as jnp
from jax import lax
from jax.experimental import pallas as pl
from jax.experimental.pallas import tpu as pltpu
from jax.experimental.pallas import tpu_sc as plsc

_M = 2048
_N = 4
_K = 64
_D_IN = 2048
_TC = _M * _N
_B = 32
_NT = 16
_CH = _TC // _NT
_NV = _CH // 16
_NEG = -3.0e38
_BIGI = 2 ** 30
_RETRY = 12


def _mm_a_body(x_ref, w_ref, b_ref, o_ref):
    acc = lax.dot_general(x_ref[...], w_ref[...], (((1,), (1,)), ((), ())),
                          preferred_element_type=jnp.float32)
    o_ref[0] = acc + b_ref[...][None, :]


def _tr_body(x_ref, o_ref):
    o_ref[...] = x_ref[...].T


def _mm_d_body(m_ref, v_ref, w_ref, b_ref, o_ref):
    cols = lax.broadcasted_iota(jnp.int32, (_B, _M), 1)
    g = jnp.where(cols == m_ref[...][:, None], v_ref[...][:, None], 0.0)
    acc = lax.dot_general(g, w_ref[...], (((1,), (1,)), ((), ())),
                          preferred_element_type=jnp.float32)
    o_ref[...] = acc + b_ref[...][None, :]


def _io16():
    return lax.broadcasted_iota(jnp.int32, (16,), 0)


def _bc_f(x):
    return jnp.full((16,), x, dtype=jnp.float32)


def _bc_i(x):
    return jnp.full((16,), x, dtype=jnp.int32)


def _shuf(v, idx):
    return v.at[idx].get(mode="promise_in_bounds")


def _rmax(v):
    io = _io16()
    for s in (8, 4, 2, 1):
        v = jnp.maximum(v, _shuf(v, (io + s) & 15))
    return v


def _rmin(v):
    return -_rmax(-v)


def _rsum(v):
    io = _io16()
    for s in (8, 4, 2, 1):
        v = v + _shuf(v, (io + s) & 15)
    return v


def _sc_body(wbt_ref, a_ref, bb_ref,
             m_out, v_out, xb_out, st_out,
             a_v, bb_v, gidx_v, col_v, sig_v, pi_v,
             w16_a, w16_b, w16_c, w16_d, w16_t,
             rf, ri, rs, rl, rt, rt3, o32_i, o32_f, sm,
             sh_min, sh_av, sh_ai, sh_sv, sh_lam, sh_cnt,
             tg_1, tg_2, tg_3, tg_4,
             sem):
    t = lax.axis_index("s") + lax.axis_index("c") * _NT
    base = t * _CH
    io = _io16()
    my = pl.ds(pl.multiple_of(t * 16, 16), 16)

    @pl.when(t < _NT)
    def _run():
        cb = lax.rem(t, 4)
        pltpu.sync_copy(a_ref.at[cb], a_v)
        pltpu.sync_copy(bb_ref.at[pl.ds(base * 1, _CH)], bb_v)
        for k in range(_NV):
            gidx_v[pl.ds(16 * k, 16)] = _bc_i(base) + _bc_i(16 * k) + io

        def _poll(copies, check, republish):
            sm[0] = 0

            def it(r, c):
                @pl.when(sm[0] == 0)
                def _p():
                    republish()
                    copies()
                    w16_d[...] = check()
                    sm[0] = w16_d[...][8]
                return c
            lax.fori_loop(0, _RETRY, it, jnp.int32(0))

        def _chk(buf, tag):
            okv = _bc_i(1)
            for q in range(_NT):
                row = buf[pl.ds(16 * q, 16)]
                okv = okv & jnp.where(row == tag, 1, 0)
            return okv

        def step(i, carry):
            jp, vp, m0, m1, v0, v1 = carry
            s_f = jnp.where(vp > 0.0, _bc_f(1.0), _bc_f(0.0))
            tgb = 4 * i

            w16_d[...] = jp
            jp_s = w16_d[...][0]
            pltpu.sync_copy(
                wbt_ref.at[jp_s, pl.ds(pl.multiple_of(base * 1, 128), _CH)],
                col_v)

            mn = _bc_f(3.0e38)
            for k in range(_NV):
                sl = pl.ds(16 * k, 16)
                af = pl.ds(pl.multiple_of(i * _CH + 16 * k, 16), 16)
                sg = a_v[af] + s_f * col_v[sl] + bb_v[sl]
                sig_v[sl] = sg
                mn = jnp.minimum(mn, sg)
            lmin = _rmin(mn)
            tag1 = jnp.asarray(tgb + 1, jnp.int32)

            def pub1():
                w16_a[...] = lmin
                pltpu.sync_copy(w16_a, sh_min.at[my])
                w16_t[...] = _bc_i(0) + tag1
                pltpu.sync_copy(w16_t, tg_1.at[my])

            _poll(lambda: pltpu.sync_copy(tg_1, rt),
                  lambda: _chk(rt, tag1), pub1)
            pltpu.sync_copy(sh_min, rf)
            gmn = _bc_f(3.0e38)
            for q in range(_NT):
                gmn = jnp.minimum(gmn, rf[pl.ds(16 * q, 16)])
            smin = _rmin(gmn)

            bv = _bc_f(_NEG)
            bi = _bc_i(_BIGI)
            bs = _bc_f(0.0)
            for k in range(_NV):
                sl = pl.ds(16 * k, 16)
                gi = gidx_v[sl]
                sg = sig_v[sl]
                damp = jnp.where(gi == jp, 1.0 - vp, _bc_f(1.0))
                pv = (sg - smin + 1.0) * damp
                pi_v[sl] = pv
                take = pv > bv
                bv = jnp.where(take, pv, bv)
                bi = jnp.where(take, gi, bi)
                bs = jnp.where(take, sg, bs)
            mv = _rmax(bv)
            jloc = _rmin(jnp.where(bv == mv, bi, _bc_i(_BIGI)))
            sloc = _rmax(jnp.where(bi == jloc, bs, _bc_f(_NEG)))
            tag2 = jnp.asarray(tgb + 2, jnp.int32)

            def pub2():
                w16_a[...] = mv
                pltpu.sync_copy(w16_a, sh_av.at[my])
                w16_b[...] = jloc
                pltpu.sync_copy(w16_b, sh_ai.at[my])
                w16_c[...] = sloc
                pltpu.sync_copy(w16_c, sh_sv.at[my])
                w16_t[...] = _bc_i(0) + tag2
                pltpu.sync_copy(w16_t, tg_2.at[my])

            _poll(lambda: pltpu.sync_copy(tg_2, rt),
                  lambda: _chk(rt, tag2), pub2)
            pltpu.sync_copy(sh_av, rf)
            pltpu.sync_copy(sh_ai, ri)
            pltpu.sync_copy(sh_sv, rs)
            gv = _bc_f(_NEG)
            gj = _bc_i(_BIGI)
            gs = _bc_f(0.0)
            for q in range(_NT):
                vr = rf[pl.ds(16 * q, 16)]
                ir = ri[pl.ds(16 * q, 16)]
                sr = rs[pl.ds(16 * q, 16)]
                take = (vr > gv) | ((vr == gv) & (ir < gj))
                gv = jnp.where(take, vr, gv)
                gj = jnp.where(take, ir, gj)
                gs = jnp.where(take, sr, gs)
            w16_d[...] = gj
            js = w16_d[...][0]
            sig_j = gs

            p = lax.rem(js, _M)
            own_p = lax.div(p, _CH // 4)
            offg = 4 * p - own_p * _CH
            tag3 = jnp.asarray(tgb + 3, jnp.int32)

            def pub3():
                @pl.when(t == own_p)
                def _pl():
                    kb = pl.multiple_of(lax.div(offg, 16) * 16, 16)
                    v = pi_v[pl.ds(kb, 16)]
                    r0 = offg - kb
                    msk = (io >= r0) & (io < r0 + 4)
                    w16_a[...] = _rmax(jnp.where(msk, v, _bc_f(_NEG)))
                    pltpu.sync_copy(w16_a, sh_lam.at[my])
                    w16_t[...] = _bc_i(0) + tag3
                    pltpu.sync_copy(w16_t, tg_3.at[my])

            rowsl = pl.ds(pl.multiple_of(own_p * 16, 16), 16)

            def cp3():
                pltpu.sync_copy(tg_3.at[rowsl], rt3)

            def ck3():
                return jnp.where(rt3[...] == tag3, 1, 0)
            _poll(cp3, ck3, pub3)
            pltpu.sync_copy(sh_lam.at[rowsl], rl)
            lam_p = rl[...]

            cnt = _bc_i(0)
            lane0 = (io & 3) == 0
            for k in range(_NV):
                sl = pl.ds(16 * k, 16)
                v = pi_v[sl]
                m1x = jnp.maximum(v, _shuf(v, io ^ 1))
                gm = jnp.maximum(m1x, _shuf(m1x, io ^ 2))
                ggi = lax.div(gidx_v[sl], _N)
                cnt = cnt + jnp.where(lane0 & (gm > lam_p), 1, 0)
                cnt = cnt + jnp.where(lane0 & (gm == lam_p) & (ggi < p), 1, 0)
            lcnt = _rsum(cnt)
            tag4 = jnp.asarray(tgb + 4, jnp.int32)

            def pub4():
                w16_b[...] = lcnt
                pltpu.sync_copy(w16_b, sh_cnt.at[my])
                w16_t[...] = _bc_i(0) + tag4
                pltpu.sync_copy(w16_t, tg_4.at[my])

            _poll(lambda: pltpu.sync_copy(tg_4, rt),
                  lambda: _chk(rt, tag4), pub4)
            pltpu.sync_copy(sh_cnt, ri)
            tot = _bc_i(0)
            for q in range(_NT):
                tot = tot + ri[pl.ds(16 * q, 16)]
            gate = tot < _K

            z = jnp.exp(2.0 * sig_j)
            tanh_v = 1.0 - 2.0 / (z + 1.0)
            y_vec = jnp.where(gate, tanh_v, _bc_f(0.0))
            v_new = jnp.maximum(y_vec, _bc_f(0.0))
            mstar = lax.div(gj, _N)

            l0 = jnp.where(i < 16, i, 99)
            l1 = jnp.where(i < 16, 99, i - 16)
            up0 = io == l0
            up1 = io == l1
            m0n = jnp.where(up0, mstar, m0)
            m1n = jnp.where(up1, mstar, m1)
            v0n = jnp.where(up0, v_new, v0)
            v1n = jnp.where(up1, v_new, v1)
            return gj, v_new, m0n, m1n, v0n, v1n

        init = (_bc_i(0), _bc_f(0.0), _bc_i(0), _bc_i(0),
                _bc_f(0.0), _bc_f(0.0))
        jl, vl, m0, m1, v0, v1 = lax.fori_loop(0, _B, step, init)

        has = vl > 0.0
        for k in range(_NV):
            sl = pl.ds(16 * k, 16)
            hit = gidx_v[sl] == jl
            sig_v[sl] = jnp.where(hit, vl, _bc_f(0.0))
            pi_v[sl] = jnp.where(hit & has, _bc_f(1.0), _bc_f(0.0))
        pltpu.sync_copy(sig_v, st_out.at[pl.ds(base * 1, _CH)])
        pltpu.sync_copy(pi_v, xb_out.at[pl.ds(base * 1, _CH)])

        @pl.when(t == 0)
        def _write_mv():
            o32_i[pl.ds(0, 16)] = m0
            o32_i[pl.ds(16, 16)] = m1
            o32_f[pl.ds(0, 16)] = v0
            o32_f[pl.ds(16, 16)] = v1
            pltpu.sync_copy(o32_i, m_out)
            pltpu.sync_copy(o32_f, v_out)


def _sc_scratch():
    f32 = jnp.float32
    i32 = jnp.int32
    return [
        pltpu.VMEM((_B * _CH,), f32),
        pltpu.VMEM((_CH,), f32),
        pltpu.VMEM((_CH,), i32),
        pltpu.VMEM((_CH,), f32),
        pltpu.VMEM((_CH,), f32),
        pltpu.VMEM((_CH,), f32),
        pltpu.VMEM((16,), f32),
        pltpu.VMEM((16,), i32),
        pltpu.VMEM((16,), f32),
        pltpu.VMEM((16,), i32),
        pltpu.VMEM((16,), i32),
        pltpu.VMEM((_NT * 16,), f32),
        pltpu.VMEM((_NT * 16,), i32),
        pltpu.VMEM((_NT * 16,), f32),
        pltpu.VMEM((16,), f32),
        pltpu.VMEM((_NT * 16,), i32),
        pltpu.VMEM((16,), i32),
        pltpu.VMEM((_B,), i32),
        pltpu.VMEM((_B,), f32),
        pltpu.SMEM((8,), i32),
        pltpu.VMEM_SHARED((_NT * 16,), f32),
        pltpu.VMEM_SHARED((_NT * 16,), f32),
        pltpu.VMEM_SHARED((_NT * 16,), i32),
        pltpu.VMEM_SHARED((_NT * 16,), f32),
        pltpu.VMEM_SHARED((_NT * 16,), f32),
        pltpu.VMEM_SHARED((_NT * 16,), i32),
        pltpu.VMEM_SHARED((_NT * 16,), i32),
        pltpu.VMEM_SHARED((_NT * 16,), i32),
        pltpu.VMEM_SHARED((_NT * 16,), i32),
        pltpu.VMEM_SHARED((_NT * 16,), i32),
        pltpu.SemaphoreType.DMA,
    ]


@jax.jit
def kernel(batch_x, W_a, b_a, W_b, b_b, W_d, b_d):
    f32 = jnp.float32
    a_blk = pl.pallas_call(
        _mm_a_body,
        grid=(4,),
        in_specs=[
            pl.BlockSpec((_B, _D_IN), lambda c: (0, 0)),
            pl.BlockSpec((_M // 4, _D_IN), lambda c: (c, 0)),
            pl.BlockSpec((_M // 4,), lambda c: (c,)),
        ],
        out_specs=pl.BlockSpec((1, _B, _M // 4), lambda c: (c, 0, 0)),
        out_shape=jax.ShapeDtypeStruct((4, _B, _M // 4), f32),
    )(batch_x, W_a, b_a)

    mesh = plsc.VectorSubcoreMesh(core_axis_name="c", subcore_axis_name="s")
    sc = pl.kernel(
        _sc_body,
        out_type=(
            jax.ShapeDtypeStruct((_B,), jnp.int32),
            jax.ShapeDtypeStruct((_B,), f32),
            jax.ShapeDtypeStruct((_TC,), f32),
            jax.ShapeDtypeStruct((_TC,), f32),
        ),
        mesh=mesh,
        scratch_types=_sc_scratch(),
    )
    wbt = pl.pallas_call(
        _tr_body,
        grid=(16, 16),
        in_specs=[pl.BlockSpec((512, 512), lambda i, j: (i, j))],
        out_specs=pl.BlockSpec((512, 512), lambda i, j: (j, i)),
        out_shape=jax.ShapeDtypeStruct((_TC, _TC), f32),
    )(W_b)
    m_idx, v_val, xb, st = sc(wbt, a_blk.reshape(4, _B * _CH), b_b)

    preds = pl.pallas_call(
        _mm_d_body,
        grid=(4,),
        in_specs=[
            pl.BlockSpec((_B,), lambda c: (0,)),
            pl.BlockSpec((_B,), lambda c: (0,)),
            pl.BlockSpec((_D_IN // 4, _M), lambda c: (c, 0)),
            pl.BlockSpec((_D_IN // 4,), lambda c: (c,)),
        ],
        out_specs=pl.BlockSpec((_B, _D_IN // 4), lambda c: (0, c)),
        out_shape=jax.ShapeDtypeStruct((_B, _D_IN), f32),
    )(m_idx, v_val, W_d, b_d)

    st2 = st.reshape(1, _TC)
    return preds, xb.reshape(1, _TC), st2, st2


# speedup vs baseline: 1.3417x; 1.3417x over previous
import jax
import jax.numpy as jnp
from jax import lax
from jax.experimental import pallas as pl
from jax.experimental.pallas import tpu as pltpu
from jax.experimental.pallas import tpu_sc as plsc

_M = 2048
_N = 4
_K = 64
_D_IN = 2048
_TC = _M * _N
_B = 32
_NT = 16
_CH = _TC // _NT
_NV = _CH // 16
_NEG = -3.0e38
_BIGI = 2 ** 30
_RETRY = 12


def _mm_a_body(x_ref, w_ref, b_ref, o_ref):
    acc = lax.dot_general(x_ref[...], w_ref[...], (((1,), (1,)), ((), ())),
                          preferred_element_type=jnp.float32)
    o_ref[0] = acc + b_ref[...][None, :]


def _mm_d_body(m_ref, v_ref, w_ref, b_ref, o_ref):
    cols = lax.broadcasted_iota(jnp.int32, (_B, _M), 1)
    g = jnp.where(cols == m_ref[...][:, None], v_ref[...][:, None], 0.0)
    acc = lax.dot_general(g, w_ref[...], (((1,), (1,)), ((), ())),
                          preferred_element_type=jnp.float32)
    o_ref[...] = acc + b_ref[...][None, :]


def _io16():
    return lax.broadcasted_iota(jnp.int32, (16,), 0)


def _bc_f(x):
    return jnp.full((16,), x, dtype=jnp.float32)


def _bc_i(x):
    return jnp.full((16,), x, dtype=jnp.int32)


def _shuf(v, idx):
    return v.at[idx].get(mode="promise_in_bounds")


def _rmax(v):
    io = _io16()
    for s in (8, 4, 2, 1):
        v = jnp.maximum(v, _shuf(v, (io + s) & 15))
    return v


def _rmin(v):
    return -_rmax(-v)


def _rsum(v):
    io = _io16()
    for s in (8, 4, 2, 1):
        v = v + _shuf(v, (io + s) & 15)
    return v


def _sc_body(wbt_ref, a_ref, bb_ref,
             m_out, v_out, xb_out, st_out,
             a_v, bb_v, gidx_v, col_v, sig_v, pi_v, gm_v,
             w16_a, w16_b, w16_c, w16_d, w16_t,
             rf, ri, rs, rg, rt, o32_i, o32_f, sm,
             sh_min, sh_av, sh_ai, sh_sv, sh_gm,
             tg_1, tg_2,
             sem):
    t = lax.axis_index("s") + lax.axis_index("c") * _NT
    base = t * _CH
    io = _io16()
    my = pl.ds(pl.multiple_of(t * 16, 16), 16)

    @pl.when(t < _NT)
    def _run():
        cb = lax.rem(t, 4)
        pltpu.sync_copy(a_ref.at[cb], a_v)
        pltpu.sync_copy(bb_ref.at[pl.ds(base * 1, _CH)], bb_v)
        for k in range(_NV):
            gidx_v[pl.ds(16 * k, 16)] = _bc_i(base) + _bc_i(16 * k) + io

        def _poll(copies, check, republish):
            sm[0] = 0

            def it(r, c):
                @pl.when(sm[0] == 0)
                def _p():
                    republish()
                    copies()
                    w16_d[...] = check()
                    sm[0] = w16_d[...][8]
                return c
            lax.fori_loop(0, _RETRY, it, jnp.int32(0))

        def _chk(buf, tag):
            okv = _bc_i(1)
            for q in range(_NT):
                row = buf[pl.ds(16 * q, 16)]
                okv = okv & jnp.where(row == tag, 1, 0)
            return okv

        def step(i, carry):
            jp, vp, m0, m1, v0, v1 = carry
            s_f = jnp.where(vp > 0.0, _bc_f(1.0), _bc_f(0.0))
            tgb = 4 * i

            w16_d[...] = jp
            jp_s = w16_d[...][0]
            pltpu.sync_copy(
                wbt_ref.at[jp_s, pl.ds(pl.multiple_of(base * 1, 128), _CH)],
                col_v)

            mn = _bc_f(3.0e38)
            for k in range(_NV):
                sl = pl.ds(16 * k, 16)
                af = pl.ds(pl.multiple_of(i * _CH + 16 * k, 16), 16)
                sg = a_v[af] + s_f * col_v[sl] + bb_v[sl]
                sig_v[sl] = sg
                mn = jnp.minimum(mn, sg)
            lmin = _rmin(mn)
            tag1 = jnp.asarray(tgb + 1, jnp.int32)

            def pub1():
                w16_a[...] = lmin
                pltpu.sync_copy(w16_a, sh_min.at[my])
                w16_t[...] = _bc_i(0) + tag1
                pltpu.sync_copy(w16_t, tg_1.at[my])

            _poll(lambda: pltpu.sync_copy(tg_1, rt),
                  lambda: _chk(rt, tag1), pub1)
            pltpu.sync_copy(sh_min, rf)
            gmn = _bc_f(3.0e38)
            for q in range(_NT):
                gmn = jnp.minimum(gmn, rf[pl.ds(16 * q, 16)])
            smin = _rmin(gmn)

            bv = _bc_f(_NEG)
            bi = _bc_i(_BIGI)
            bs = _bc_f(0.0)
            for k in range(_NV):
                sl = pl.ds(16 * k, 16)
                gi = gidx_v[sl]
                sg = sig_v[sl]
                damp = jnp.where(gi == jp, 1.0 - vp, _bc_f(1.0))
                pv = (sg - smin + 1.0) * damp
                pi_v[sl] = pv
                take = pv > bv
                bv = jnp.where(take, pv, bv)
                bi = jnp.where(take, gi, bi)
                bs = jnp.where(take, sg, bs)
            mv = _rmax(bv)
            jloc = _rmin(jnp.where(bv == mv, bi, _bc_i(_BIGI)))
            sloc = _rmax(jnp.where(bi == jloc, bs, _bc_f(_NEG)))
            cperm = (io * 4) & 15
            for q in range(_NV // 4):
                outs = []
                for u in range(4):
                    v = pi_v[pl.ds(16 * (4 * q + u), 16)]
                    m1x = jnp.maximum(v, _shuf(v, io ^ 1))
                    gmv = jnp.maximum(m1x, _shuf(m1x, io ^ 2))
                    outs.append(_shuf(gmv, cperm))
                cc = jnp.where(io < 4, outs[0],
                               jnp.where(io < 8, outs[1],
                                         jnp.where(io < 12, outs[2], outs[3])))
                gm_v[pl.ds(16 * q, 16)] = cc
            tag2 = jnp.asarray(tgb + 2, jnp.int32)

            def pub2():
                w16_a[...] = mv
                pltpu.sync_copy(w16_a, sh_av.at[my])
                w16_b[...] = jloc
                pltpu.sync_copy(w16_b, sh_ai.at[my])
                w16_c[...] = sloc
                pltpu.sync_copy(w16_c, sh_sv.at[my])
                pltpu.sync_copy(
                    gm_v, sh_gm.at[pl.ds(pl.multiple_of(t * 128, 128), 128)])
                w16_t[...] = _bc_i(0) + tag2
                pltpu.sync_copy(w16_t, tg_2.at[my])

            _poll(lambda: pltpu.sync_copy(tg_2, rt),
                  lambda: _chk(rt, tag2), pub2)
            pltpu.sync_copy(sh_av, rf)
            pltpu.sync_copy(sh_ai, ri)
            pltpu.sync_copy(sh_sv, rs)
            pltpu.sync_copy(sh_gm, rg)
            gv = _bc_f(_NEG)
            gj = _bc_i(_BIGI)
            gs = _bc_f(0.0)
            for q in range(_NT):
                vr = rf[pl.ds(16 * q, 16)]
                ir = ri[pl.ds(16 * q, 16)]
                sr = rs[pl.ds(16 * q, 16)]
                take = (vr > gv) | ((vr == gv) & (ir < gj))
                gv = jnp.where(take, vr, gv)
                gj = jnp.where(take, ir, gj)
                gs = jnp.where(take, sr, gs)
            w16_d[...] = gj
            js = w16_d[...][0]
            sig_j = gs

            p = lax.rem(js, _M)
            pv16 = pl.ds(pl.multiple_of(lax.div(p, 16) * 16, 16), 16)
            lamrow = rg[pv16]
            lam_p = _rmax(jnp.where(io == lax.rem(p, 16), lamrow,
                                    _bc_f(_NEG)))
            tot = _bc_i(0)
            for q in range(_M // 16):
                gmr = rg[pl.ds(16 * q, 16)]
                ggi = _bc_i(16 * q) + io
                tot = tot + jnp.where(gmr > lam_p, 1, 0)
                tot = tot + jnp.where((gmr == lam_p) & (ggi < p), 1, 0)
            tot = _rsum(tot)
            gate = tot < _K

            z = jnp.exp(2.0 * sig_j)
            tanh_v = 1.0 - 2.0 / (z + 1.0)
            y_vec = jnp.where(gate, tanh_v, _bc_f(0.0))
            v_new = jnp.maximum(y_vec, _bc_f(0.0))
            mstar = lax.div(gj, _N)

            l0 = jnp.where(i < 16, i, 99)
            l1 = jnp.where(i < 16, 99, i - 16)
            up0 = io == l0
            up1 = io == l1
            m0n = jnp.where(up0, mstar, m0)
            m1n = jnp.where(up1, mstar, m1)
            v0n = jnp.where(up0, v_new, v0)
            v1n = jnp.where(up1, v_new, v1)
            return gj, v_new, m0n, m1n, v0n, v1n

        init = (_bc_i(0), _bc_f(0.0), _bc_i(0), _bc_i(0),
                _bc_f(0.0), _bc_f(0.0))
        jl, vl, m0, m1, v0, v1 = lax.fori_loop(0, _B, step, init)

        has = vl > 0.0
        for k in range(_NV):
            sl = pl.ds(16 * k, 16)
            hit = gidx_v[sl] == jl
            sig_v[sl] = jnp.where(hit, vl, _bc_f(0.0))
            pi_v[sl] = jnp.where(hit & has, _bc_f(1.0), _bc_f(0.0))
        pltpu.sync_copy(sig_v, st_out.at[pl.ds(base * 1, _CH)])
        pltpu.sync_copy(pi_v, xb_out.at[pl.ds(base * 1, _CH)])

        @pl.when(t == 0)
        def _write_mv():
            o32_i[pl.ds(0, 16)] = m0
            o32_i[pl.ds(16, 16)] = m1
            o32_f[pl.ds(0, 16)] = v0
            o32_f[pl.ds(16, 16)] = v1
            pltpu.sync_copy(o32_i, m_out)
            pltpu.sync_copy(o32_f, v_out)


def _sc_scratch():
    f32 = jnp.float32
    i32 = jnp.int32
    return [
        pltpu.VMEM((_B * _CH,), f32),
        pltpu.VMEM((_CH,), f32),
        pltpu.VMEM((_CH,), i32),
        pltpu.VMEM((_CH,), f32),
        pltpu.VMEM((_CH,), f32),
        pltpu.VMEM((_CH,), f32),
        pltpu.VMEM((_CH // 4,), f32),
        pltpu.VMEM((16,), f32),
        pltpu.VMEM((16,), i32),
        pltpu.VMEM((16,), f32),
        pltpu.VMEM((16,), i32),
        pltpu.VMEM((16,), i32),
        pltpu.VMEM((_NT * 16,), f32),
        pltpu.VMEM((_NT * 16,), i32),
        pltpu.VMEM((_NT * 16,), f32),
        pltpu.VMEM((_M,), f32),
        pltpu.VMEM((_NT * 16,), i32),
        pltpu.VMEM((_B,), i32),
        pltpu.VMEM((_B,), f32),
        pltpu.SMEM((8,), i32),
        pltpu.VMEM_SHARED((_NT * 16,), f32),
        pltpu.VMEM_SHARED((_NT * 16,), f32),
        pltpu.VMEM_SHARED((_NT * 16,), i32),
        pltpu.VMEM_SHARED((_NT * 16,), f32),
        pltpu.VMEM_SHARED((_M,), f32),
        pltpu.VMEM_SHARED((_NT * 16,), i32),
        pltpu.VMEM_SHARED((_NT * 16,), i32),
        pltpu.SemaphoreType.DMA,
    ]


@jax.jit
def kernel(batch_x, W_a, b_a, W_b, b_b, W_d, b_d):
    f32 = jnp.float32
    a_blk = pl.pallas_call(
        _mm_a_body,
        grid=(4,),
        in_specs=[
            pl.BlockSpec((_B, _D_IN), lambda c: (0, 0)),
            pl.BlockSpec((_M // 4, _D_IN), lambda c: (c, 0)),
            pl.BlockSpec((_M // 4,), lambda c: (c,)),
        ],
        out_specs=pl.BlockSpec((1, _B, _M // 4), lambda c: (c, 0, 0)),
        out_shape=jax.ShapeDtypeStruct((4, _B, _M // 4), f32),
    )(batch_x, W_a, b_a)

    mesh = plsc.VectorSubcoreMesh(core_axis_name="c", subcore_axis_name="s")
    sc = pl.kernel(
        _sc_body,
        out_type=(
            jax.ShapeDtypeStruct((_B,), jnp.int32),
            jax.ShapeDtypeStruct((_B,), f32),
            jax.ShapeDtypeStruct((_TC,), f32),
            jax.ShapeDtypeStruct((_TC,), f32),
        ),
        mesh=mesh,
        scratch_types=_sc_scratch(),
    )
    m_idx, v_val, xb, st = sc(jnp.swapaxes(W_b, 0, 1),
                              a_blk.reshape(4, _B * _CH), b_b)

    preds = pl.pallas_call(
        _mm_d_body,
        grid=(4,),
        in_specs=[
            pl.BlockSpec((_B,), lambda c: (0,)),
            pl.BlockSpec((_B,), lambda c: (0,)),
            pl.BlockSpec((_D_IN // 4, _M), lambda c: (c, 0)),
            pl.BlockSpec((_D_IN // 4,), lambda c: (c,)),
        ],
        out_specs=pl.BlockSpec((_B, _D_IN // 4), lambda c: (0, c)),
        out_shape=jax.ShapeDtypeStruct((_B, _D_IN), f32),
    )(m_idx, v_val, W_d, b_d)

    st2 = st.reshape(1, _TC)
    return preds, xb.reshape(1, _TC), st2, st2
